# baseline (speedup 1.0000x reference)
import functools

import jax
import jax.numpy as jnp
from jax import lax
from jax.experimental import pallas as pl
from jax.experimental.pallas import tpu as pltpu
from jax.experimental.pallas import tpu_sc as plsc

DIM = 64
CHUNK = 400


def _make_sc_kernel(n_rows: int, n_workers: int):
    rows_per_w = n_rows // n_workers
    n_chunks = rows_per_w // CHUNK
    mesh = plsc.VectorSubcoreMesh(core_axis_name="c", subcore_axis_name="s")

    @functools.partial(
        pl.kernel,
        mesh=mesh,
        compiler_params=pltpu.CompilerParams(use_tc_tiling_on_sc=False),
        out_type=jax.ShapeDtypeStruct((n_rows, DIM), jnp.float32),
        scratch_types=[
            pltpu.VMEM((CHUNK,), jnp.int32),
            pltpu.VMEM((CHUNK,), jnp.int32),
            pltpu.VMEM((CHUNK,), jnp.int32),
            pltpu.VMEM((CHUNK, DIM), jnp.float32),
            pltpu.SemaphoreType.DMA,
            pltpu.SemaphoreType.DMA,
            pltpu.SemaphoreType.DMA,
        ],
    )
    def sc_kernel(idx_a_hbm, idx_b_hbm, idx_c_hbm, tab_a_hbm, tab_b_hbm,
                  tab_c_hbm, pos_hbm, out_hbm,
                  idx_a, idx_b, idx_c, acc,
                  sem_a, sem_b, sem_c):
        n_cores = 2
        wid = lax.axis_index("s") * n_cores + lax.axis_index("c")
        w_base = wid * rows_per_w

        def do_chunk(ci, carry):
            base = w_base + ci * CHUNK
            sl = pl.ds(base, CHUNK)
            pltpu.sync_copy(idx_a_hbm.at[sl], idx_a)
            pltpu.sync_copy(idx_b_hbm.at[sl], idx_b)
            pltpu.sync_copy(idx_c_hbm.at[sl], idx_c)
            pltpu.sync_copy(pos_hbm, acc)
            pltpu.async_copy(tab_a_hbm.at[idx_a], acc, sem_a, add=True).wait()
            pltpu.async_copy(tab_b_hbm.at[idx_b], acc, sem_b, add=True).wait()
            pltpu.async_copy(tab_c_hbm.at[idx_c], acc, sem_c, add=True).wait()
            pltpu.sync_copy(acc, out_hbm.at[sl])
            return carry

        lax.fori_loop(0, n_chunks, do_chunk, 0)

    return sc_kernel


def kernel(feat_item, feat_category, feat_brand, positions,
           table_item, table_category, table_brand, table_position):
    batch, pos_len = feat_item.shape
    n_rows = batch * pos_len

    idx_a = feat_item.reshape(n_rows)
    idx_b = feat_category.reshape(n_rows)
    idx_c = feat_brand.reshape(n_rows)

    pos_rows = jnp.take(table_position, positions, axis=0)
    pos_block = jnp.tile(pos_rows, (CHUNK // pos_len, 1))

    sc = _make_sc_kernel(n_rows, 32)
    out = sc(idx_a, idx_b, idx_c, table_item, table_category,
             table_brand, pos_block)
    return out.reshape(batch, pos_len, DIM)


# speedup vs baseline: 1.1249x; 1.1249x over previous
import functools

import jax
import jax.numpy as jnp
from jax import lax
from jax.experimental import pallas as pl
from jax.experimental.pallas import tpu as pltpu
from jax.experimental.pallas import tpu_sc as plsc

DIM = 64
CHUNK = 800
NBUF = 2


def _make_sc_kernel(n_rows: int, n_workers: int):
    rows_per_w = n_rows // n_workers
    n_chunks = rows_per_w // CHUNK
    mesh = plsc.VectorSubcoreMesh(core_axis_name="c", subcore_axis_name="s")

    @functools.partial(
        pl.kernel,
        mesh=mesh,
        compiler_params=pltpu.CompilerParams(use_tc_tiling_on_sc=False),
        out_type=jax.ShapeDtypeStruct((n_rows, DIM), jnp.float32),
        scratch_types=[
            [pltpu.VMEM((CHUNK,), jnp.int32)] * NBUF,
            [pltpu.VMEM((CHUNK,), jnp.int32)] * NBUF,
            [pltpu.VMEM((CHUNK,), jnp.int32)] * NBUF,
            [pltpu.VMEM((CHUNK, DIM), jnp.float32)] * NBUF,
            [pltpu.SemaphoreType.DMA] * NBUF,
            [pltpu.SemaphoreType.DMA] * NBUF,
        ],
    )
    def sc_kernel(idx_a_hbm, idx_b_hbm, idx_c_hbm, tab_a_hbm, tab_b_hbm,
                  tab_c_hbm, pos_hbm, out_hbm,
                  idx_a, idx_b, idx_c, acc, sem_g, sem_o):
        n_cores = 2
        wid = lax.axis_index("s") * n_cores + lax.axis_index("c")
        w_base = wid * rows_per_w

        gathers = [None] * NBUF
        out_cp = [None] * NBUF

        def chunk_slice(c):
            return pl.ds(w_base + c * CHUNK, CHUNK)

        for c in range(n_chunks):
            p = c % NBUF
            if out_cp[p] is not None:
                out_cp[p].wait()
            sl = chunk_slice(c)
            pltpu.sync_copy(idx_a_hbm.at[sl], idx_a[p])
            pltpu.sync_copy(idx_b_hbm.at[sl], idx_b[p])
            pltpu.sync_copy(idx_c_hbm.at[sl], idx_c[p])
            pltpu.sync_copy(pos_hbm, acc[p])
            gathers[p] = [
                pltpu.async_copy(tab_a_hbm.at[idx_a[p]], acc[p], sem_g[p], add=True),
                pltpu.async_copy(tab_b_hbm.at[idx_b[p]], acc[p], sem_g[p], add=True),
                pltpu.async_copy(tab_c_hbm.at[idx_c[p]], acc[p], sem_g[p], add=True),
            ]
            if c >= 1:
                q = (c - 1) % NBUF
                for d in gathers[q]:
                    d.wait()
                out_cp[q] = pltpu.async_copy(acc[q], out_hbm.at[chunk_slice(c - 1)], sem_o[q])
        last = (n_chunks - 1) % NBUF
        for d in gathers[last]:
            d.wait()
        pltpu.async_copy(acc[last], out_hbm.at[chunk_slice(n_chunks - 1)], sem_o[last]).wait()
        out_cp[(n_chunks - 2) % NBUF].wait()

    return sc_kernel


def kernel(feat_item, feat_category, feat_brand, positions,
           table_item, table_category, table_brand, table_position):
    batch, pos_len = feat_item.shape
    n_rows = batch * pos_len

    idx_a = feat_item.reshape(n_rows)
    idx_b = feat_category.reshape(n_rows)
    idx_c = feat_brand.reshape(n_rows)

    pos_rows = jnp.take(table_position, positions, axis=0)
    pos_block = jnp.tile(pos_rows, (CHUNK // pos_len, 1))

    sc = _make_sc_kernel(n_rows, 32)
    out = sc(idx_a, idx_b, idx_c, table_item, table_category,
             table_brand, pos_block)
    return out.reshape(batch, pos_len, DIM)
